# baseline (speedup 1.0000x reference)
import functools

import jax
import jax.numpy as jnp
from jax import lax
from jax.experimental import pallas as pl
from jax.experimental.pallas import tpu as pltpu
from jax.experimental.pallas import tpu_sc as plsc

N_DATA = 100000
N_HID = 10000
N_HID_PAD = 10240
E = 320000
CH = 128
C_OUT = 80


_F32 = jnp.float32


def _dot(a, b):
    return jnp.dot(a, b, preferred_element_type=_F32)


def _k1a_body(x0_ref, x1_ref, c_ref, tr_ref, w0_ref, w1_ref, w2_ref,
              wm_ref, h_ref, msg_ref):
    x0 = x0_ref[0, 0, 0]
    x1 = x1_ref[0, 0, 0]
    c = c_ref[...]
    f12 = jnp.concatenate([jnp.sin(c), jnp.cos(c), tr_ref[...]], axis=-1)
    h = (_dot(x0, w0_ref[...]) + _dot(x1, w1_ref[...])
         + _dot(f12, w2_ref[...]))
    h_ref[...] = h
    msg_ref[...] = jnp.maximum(_dot(h, wm_ref[...]), 0.0)


def _k1a(x, coords, tr, w0, w1, w2, w_msg):
    B = 2000
    n = N_DATA // B
    V = x.shape[-1]
    return pl.pallas_call(
        _k1a_body,
        grid=(n,),
        in_specs=[
            pl.BlockSpec((1, 1, 1, B, V), lambda i: (0, 0, 0, i, 0)),
            pl.BlockSpec((1, 1, 1, B, V), lambda i: (0, 1, 0, i, 0)),
            pl.BlockSpec((B, 2), lambda i: (i, 0)),
            pl.BlockSpec((B, tr.shape[1]), lambda i: (i, 0)),
            pl.BlockSpec(w0.shape, lambda i: (0, 0)),
            pl.BlockSpec(w1.shape, lambda i: (0, 0)),
            pl.BlockSpec(w2.shape, lambda i: (0, 0)),
            pl.BlockSpec(w_msg.shape, lambda i: (0, 0)),
        ],
        out_specs=[
            pl.BlockSpec((B, CH), lambda i: (i, 0)),
            pl.BlockSpec((B, CH), lambda i: (i, 0)),
        ],
        out_shape=[
            jax.ShapeDtypeStruct((N_DATA, CH), _F32),
            jax.ShapeDtypeStruct((N_DATA, CH), _F32),
        ],
    )(x, x, coords, tr, w0, w1, w2, w_msg)


def _k1b_body(h_ref, wd_ref, hd_ref):
    hd_ref[...] = _dot(jnp.maximum(h_ref[...], 0.0), wd_ref[...])


def _k1b(h, w_dec_dst):
    B = 2000
    n = N_DATA // B
    return pl.pallas_call(
        _k1b_body,
        grid=(n,),
        in_specs=[
            pl.BlockSpec((B, CH), lambda i: (i, 0)),
            pl.BlockSpec(w_dec_dst.shape, lambda i: (0, 0)),
        ],
        out_specs=pl.BlockSpec((B, CH), lambda i: (i, 0)),
        out_shape=jax.ShapeDtypeStruct((N_DATA, CH), _F32),
    )(h, w_dec_dst)


def _k2_body(fh_ref, wdst_ref, parts_ref, wu_ref, wm0_ref, xl_ref, m0_ref):
    agg = parts_ref[0] + parts_ref[1]
    hdst = _dot(fh_ref[...], wdst_ref[...])
    xl = jnp.maximum(hdst + _dot(agg, wu_ref[...]), 0.0)
    xl_ref[...] = xl
    m0_ref[...] = jnp.maximum(_dot(xl, wm0_ref[...]), 0.0)


def _k2(fh, w_dst, parts, w_upd, w_msg0):
    B = 1024
    n = N_HID_PAD // B
    return pl.pallas_call(
        _k2_body,
        grid=(n,),
        in_specs=[
            pl.BlockSpec((B, fh.shape[1]), lambda i: (i, 0)),
            pl.BlockSpec(w_dst.shape, lambda i: (0, 0)),
            pl.BlockSpec((2, B, CH), lambda i: (0, i, 0)),
            pl.BlockSpec(w_upd.shape, lambda i: (0, 0)),
            pl.BlockSpec(w_msg0.shape, lambda i: (0, 0)),
        ],
        out_specs=[
            pl.BlockSpec((B, CH), lambda i: (i, 0)),
            pl.BlockSpec((B, CH), lambda i: (i, 0)),
        ],
        out_shape=[
            jax.ShapeDtypeStruct((N_HID_PAD, CH), _F32),
            jax.ShapeDtypeStruct((N_HID_PAD, CH), _F32),
        ],
    )(fh, w_dst, parts, w_upd, w_msg0)


def _k3_body(h_ref, parts_ref, wu_ref, wm_ref, h1_ref, m1_ref):
    agg = parts_ref[0] + parts_ref[1]
    h1 = h_ref[...] + jnp.maximum(_dot(agg, wu_ref[...]), 0.0)
    h1_ref[...] = h1
    m1_ref[...] = jnp.maximum(_dot(h1, wm_ref[...]), 0.0)


def _k3(h, parts, w_upd, w_msg):
    B = 1024
    n = N_HID_PAD // B
    return pl.pallas_call(
        _k3_body,
        grid=(n,),
        in_specs=[
            pl.BlockSpec((B, CH), lambda i: (i, 0)),
            pl.BlockSpec((2, B, CH), lambda i: (0, i, 0)),
            pl.BlockSpec(w_upd.shape, lambda i: (0, 0)),
            pl.BlockSpec(w_msg.shape, lambda i: (0, 0)),
        ],
        out_specs=[
            pl.BlockSpec((B, CH), lambda i: (i, 0)),
            pl.BlockSpec((B, CH), lambda i: (i, 0)),
        ],
        out_shape=[
            jax.ShapeDtypeStruct((N_HID_PAD, CH), _F32),
            jax.ShapeDtypeStruct((N_HID_PAD, CH), _F32),
        ],
    )(h, parts, w_upd, w_msg)


def _k4_body(h1_ref, xl_ref, parts_ref, wu_ref, wmd_ref, md_ref):
    agg = parts_ref[0] + parts_ref[1]
    xlp = h1_ref[...] + jnp.maximum(_dot(agg, wu_ref[...]), 0.0) + xl_ref[...]
    md_ref[...] = jnp.maximum(_dot(xlp, wmd_ref[...]), 0.0)


def _k4(h1, xl, parts, w_upd, w_msg_dec):
    B = 1024
    n = N_HID_PAD // B
    return pl.pallas_call(
        _k4_body,
        grid=(n,),
        in_specs=[
            pl.BlockSpec((B, CH), lambda i: (i, 0)),
            pl.BlockSpec((B, CH), lambda i: (i, 0)),
            pl.BlockSpec((2, B, CH), lambda i: (0, i, 0)),
            pl.BlockSpec(w_upd.shape, lambda i: (0, 0)),
            pl.BlockSpec(w_msg_dec.shape, lambda i: (0, 0)),
        ],
        out_specs=pl.BlockSpec((B, CH), lambda i: (i, 0)),
        out_shape=jax.ShapeDtypeStruct((N_HID_PAD, CH), _F32),
    )(h1, xl, parts, w_upd, w_msg_dec)


def _k5_body(hd_ref, a_ref, wu_ref, wo_ref, x1_ref, out_ref):
    hf = jnp.maximum(hd_ref[...] + _dot(a_ref[...], wu_ref[...]), 0.0)
    out_ref[...] = _dot(hf, wo_ref[...]) + x1_ref[0, 0, 0]


def _k5(hd, a, w_upd, w_out, x):
    B = 2000
    n = N_DATA // B
    V = x.shape[-1]
    return pl.pallas_call(
        _k5_body,
        grid=(n,),
        in_specs=[
            pl.BlockSpec((B, CH), lambda i: (i, 0)),
            pl.BlockSpec((B, CH), lambda i: (i, 0)),
            pl.BlockSpec(w_upd.shape, lambda i: (0, 0)),
            pl.BlockSpec(w_out.shape, lambda i: (0, 0)),
            pl.BlockSpec((1, 1, 1, B, V), lambda i: (0, 1, 0, i, 0)),
        ],
        out_specs=pl.BlockSpec((B, C_OUT), lambda i: (i, 0)),
        out_shape=jax.ShapeDtypeStruct((N_DATA, C_OUT), _F32),
    )(hd, a, w_upd, w_out, x)


_EB = 128
_CPT = 80
_PP = 16
_EPAD = 32 * _CPT * _EB
_NHA = 10368
_TPH = _NHA // 16


def _sc_agg_hidden(p_hbm, src2d, dst2d, zeros_hid):
    mesh = plsc.VectorSubcoreMesh(core_axis_name="c", subcore_axis_name="s")

    @functools.partial(
        pl.kernel,
        out_type=jax.ShapeDtypeStruct((2, _NHA, CH), _F32),
        mesh=mesh,
        compiler_params=pltpu.CompilerParams(needs_layout_passes=False),
        scratch_types=[
            pltpu.VMEM((_PP, _EB), jnp.int32),
            pltpu.VMEM((_PP, _EB), jnp.int32),
            pltpu.VMEM((_EB, CH), _F32),
            pltpu.VMEM((_EB, CH), _F32),
            pltpu.VMEM_SHARED((_NHA, CH), _F32),
            pltpu.SemaphoreType.DMA,
            pltpu.SemaphoreType.DMA,
        ],
    )
    def k(p_ref, src_ref, dst_ref, z_ref, out_ref,
          srcv, dstv, rows0, rows1, acc, sem0, sem1):
        c = lax.axis_index("c")
        s = lax.axis_index("s")
        wid = c * 16 + s

        pltpu.sync_copy(z_ref, acc.at[pl.ds(s * _TPH, _TPH)])
        plsc.subcore_barrier()

        def piece(p, _):
            base = wid * _CPT + p * _PP
            pltpu.sync_copy(src_ref.at[pl.ds(base, _PP)], srcv)
            pltpu.sync_copy(dst_ref.at[pl.ds(base, _PP)], dstv)

            pltpu.async_copy(p_ref.at[srcv.at[0]], rows0, sem0)

            def pair(t2, _):
                jj = 2 * t2
                pltpu.make_async_copy(p_ref.at[srcv.at[0]], rows0,
                                      sem0).wait()
                pltpu.async_copy(p_ref.at[srcv.at[jj + 1]], rows1, sem1)
                pltpu.sync_copy(rows0, acc.at[dstv.at[jj]], add=True)
                pltpu.make_async_copy(p_ref.at[srcv.at[0]], rows1,
                                      sem1).wait()

                @pl.when(jj + 2 < _PP)
                def _():
                    pltpu.async_copy(
                        p_ref.at[srcv.at[jnp.minimum(jj + 2, _PP - 1)]],
                        rows0, sem0)

                pltpu.sync_copy(rows1, acc.at[dstv.at[jj + 1]], add=True)
                return 0

            lax.fori_loop(0, _PP // 2, pair, 0)
            return 0

        lax.fori_loop(0, _CPT // _PP, piece, 0)
        plsc.subcore_barrier()
        pltpu.sync_copy(acc.at[pl.ds(s * _TPH, _TPH)],
                        out_ref.at[c, pl.ds(s * _TPH, _TPH)])

    return k(p_hbm, src2d, dst2d, zeros_hid)


_CDST = 10240
_CPAD = 10368
_NCHUNK = 10
_SH = E // 16
_ES = 2000
_BT = 128
_CAP = 2304
_TPR = _CPAD // 16
_SHIFT = 14
_MASKV = (1 << _SHIFT) - 1


def _sc_agg_data(p_hbm, src_hbm, dst_hbm, zeros_dat):
    mesh = plsc.VectorSubcoreMesh(core_axis_name="c", subcore_axis_name="s")

    @functools.partial(
        pl.kernel,
        out_type=jax.ShapeDtypeStruct((_NCHUNK * _CDST, CH), _F32),
        mesh=mesh,
        compiler_params=pltpu.CompilerParams(needs_layout_passes=False),
        scratch_types=[
            pltpu.VMEM((_ES,), jnp.int32),
            pltpu.VMEM((_ES,), jnp.int32),
            pltpu.VMEM((_CAP,), jnp.int32),
            pltpu.VMEM((_BT,), jnp.int32),
            pltpu.VMEM((_BT,), jnp.int32),
            pltpu.VMEM((_BT,), jnp.int32),
            pltpu.VMEM((_BT,), jnp.int32),
            pltpu.VMEM((_BT, CH), _F32),
            pltpu.VMEM((_BT, CH), _F32),
            pltpu.VMEM_SHARED((_CPAD, CH), _F32),
            pltpu.SemaphoreType.DMA,
            pltpu.SemaphoreType.DMA,
        ],
    )
    def k(p_ref, src_ref, dst_ref, z_ref, out_ref,
          stag_s, stag_d, cpk, bs0, bd0, bs1, bd1, rows0, rows1, acc,
          sem0, sem1):
        c = lax.axis_index("c")
        s = lax.axis_index("s")

        def prep(j, bs, bd, n):
            for i in range(_BT // 16):
                v = cpk[pl.ds(j * _BT + i * 16, 16)]
                sv = v & _MASKV
                dv = lax.shift_right_logical(v, _SHIFT)
                if n is not None:
                    valid = (j * _BT + i * 16 + lax.iota(jnp.int32, 16)) < n
                    sv = jnp.where(valid, sv, 0)
                    dv = jnp.where(valid, dv, _CDST)
                bs[pl.ds(i * 16, 16)] = sv
                bd[pl.ds(i * 16, 16)] = dv

        def do_batches(nb, n=None):
            @pl.when(nb > 0)
            def _():
                prep(0, bs0, bd0, n)
                pltpu.async_copy(p_ref.at[bs0], rows0, sem0)

            def pair(tt, _):
                j = 2 * tt
                pltpu.make_async_copy(p_ref.at[bs0], rows0, sem0).wait()

                @pl.when(j + 1 < nb)
                def _():
                    prep(j + 1, bs1, bd1, n)
                    pltpu.async_copy(p_ref.at[bs1], rows1, sem1)

                pltpu.sync_copy(rows0, acc.at[bd0], add=True)

                @pl.when(j + 1 < nb)
                def _():
                    pltpu.make_async_copy(p_ref.at[bs1], rows1, sem1).wait()

                    @pl.when(j + 2 < nb)
                    def _():
                        prep(j + 2, bs0, bd0, n)
                        pltpu.async_copy(p_ref.at[bs0], rows0, sem0)

                    pltpu.sync_copy(rows1, acc.at[bd1], add=True)

                return 0

            lax.fori_loop(0, (nb + 1) // 2, pair, 0)

        for ci in range(_NCHUNK // 2):
            chunk = c * (_NCHUNK // 2) + ci
            lo = chunk * _CDST

            pltpu.sync_copy(z_ref, acc.at[pl.ds(s * _TPR, _TPR)])
            plsc.subcore_barrier()

            zc = jnp.zeros((16,), jnp.int32)

            def piece(p, cnt):
                off = s * _SH + p * _ES
                pltpu.sync_copy(src_ref.at[pl.ds(off, _ES)], stag_s)
                pltpu.sync_copy(dst_ref.at[pl.ds(off, _ES)], stag_d)

                def inner(i, cnt):
                    d = stag_d[pl.ds(i * 16, 16)]
                    sv = stag_s[pl.ds(i * 16, 16)]
                    m = (d >= lo) & (d < lo + _CDST)
                    mi = m.astype(jnp.int32)
                    pos = cnt + plsc.cumsum(mi) - mi
                    idx = jnp.where(m, pos, _CAP - 1)
                    v = lax.shift_left(d - lo, _SHIFT) | sv
                    plsc.store_scatter(cpk, [idx], v)
                    return cnt + plsc.all_reduce_population_count(m)

                cnt = lax.fori_loop(0, _ES // 16, inner, cnt)
                n = jnp.max(cnt)
                nb = n // _BT
                do_batches(nb)
                for i in range(_BT // 16):
                    tv_ = cpk[pl.ds(nb * _BT + i * 16, 16)]
                    cpk[pl.ds(i * 16, 16)] = tv_
                return cnt - nb * _BT

            cnt = lax.fori_loop(0, _SH // _ES, piece, zc)
            n = jnp.max(cnt)

            do_batches((n + _BT - 1) // _BT, n)
            plsc.subcore_barrier()

            r0 = s * (_CDST // 16)
            pltpu.sync_copy(acc.at[pl.ds(r0, _CDST // 16)],
                            out_ref.at[pl.ds(chunk * _CDST + r0, _CDST // 16)])

    return k(p_hbm, src_hbm, dst_hbm, zeros_dat)


def kernel(x, coords_data, coords_hidden, trainable_data, trainable_hidden,
           W_enc_src, W_enc_dst, W_enc_msg, W_enc_upd,
           W_proc_msg_0, W_proc_upd_0, W_proc_msg_1, W_proc_upd_1,
           W_dec_dst, W_dec_msg, W_dec_upd, W_out,
           enc_src_idx, enc_dst_idx, proc_src_idx, proc_dst_idx,
           dec_src_idx, dec_dst_idx):
    b, t, e, g, v = x.shape
    latlons_hidden = jnp.concatenate(
        [jnp.sin(coords_hidden), jnp.cos(coords_hidden)], axis=-1)
    fh = jnp.concatenate([latlons_hidden, trainable_hidden], axis=-1)
    fh = jnp.pad(fh, ((0, N_HID_PAD - N_HID), (0, 0)))

    h_src, msg_enc = _k1a(x, coords_data, trainable_data,
                          W_enc_src[:v], W_enc_src[v:2 * v],
                          W_enc_src[2 * v:], W_enc_msg)
    hd = _k1b(h_src, W_dec_dst)

    zeros_hid = jnp.zeros((_TPH, CH), _F32)
    zeros_dat = jnp.zeros((_TPR, CH), _F32)

    npad = _EPAD - E
    pad_src = jnp.arange(npad, dtype=jnp.int32) % N_HID
    pad_dst = N_HID_PAD + jnp.arange(npad, dtype=jnp.int32) % (_NHA - N_HID_PAD)

    def _pad_idx(idx, padv):
        return jnp.concatenate([idx, padv]).reshape(-1, _EB)

    parts_enc = _sc_agg_hidden(
        msg_enc, _pad_idx(enc_src_idx, pad_src),
        _pad_idx(enc_dst_idx, pad_dst), zeros_hid)

    xl, msg_p0 = _k2(fh, W_enc_dst, parts_enc, W_enc_upd, W_proc_msg_0)

    src2d_p = _pad_idx(proc_src_idx, pad_src)
    dst2d_p = _pad_idx(proc_dst_idx, pad_dst)

    parts0 = _sc_agg_hidden(msg_p0, src2d_p, dst2d_p, zeros_hid)
    h1, msg_p1 = _k3(xl, parts0, W_proc_upd_0, W_proc_msg_1)

    parts1 = _sc_agg_hidden(msg_p1, src2d_p, dst2d_p, zeros_hid)
    msg_dec = _k4(h1, xl, parts1, W_proc_upd_1, W_dec_msg)

    a = _sc_agg_data(msg_dec, dec_src_idx, dec_dst_idx, zeros_dat)

    out = _k5(hd, a, W_dec_upd, W_out, x)
    return out.reshape(b, e, g, C_OUT)


# speedup vs baseline: 1.0119x; 1.0119x over previous
import functools

import jax
import jax.numpy as jnp
from jax import lax
from jax.experimental import pallas as pl
from jax.experimental.pallas import tpu as pltpu
from jax.experimental.pallas import tpu_sc as plsc

N_DATA = 100000
N_HID = 10000
N_HID_PAD = 10240
E = 320000
CH = 128
C_OUT = 80


_F32 = jnp.float32


def _dot(a, b):
    return jnp.dot(a, b, preferred_element_type=_F32)


def _k1a_body(x0_ref, x1_ref, c_ref, tr_ref, w0_ref, w1_ref, w2_ref,
              wm_ref, h_ref, msg_ref):
    x0 = x0_ref[0, 0, 0]
    x1 = x1_ref[0, 0, 0]
    c = c_ref[...]
    f12 = jnp.concatenate([jnp.sin(c), jnp.cos(c), tr_ref[...]], axis=-1)
    h = (_dot(x0, w0_ref[...]) + _dot(x1, w1_ref[...])
         + _dot(f12, w2_ref[...]))
    h_ref[...] = h
    msg_ref[...] = jnp.maximum(_dot(h, wm_ref[...]), 0.0)


def _k1a(x, coords, tr, w0, w1, w2, w_msg):
    B = 2000
    n = N_DATA // B
    V = x.shape[-1]
    return pl.pallas_call(
        _k1a_body,
        grid=(n,),
        in_specs=[
            pl.BlockSpec((1, 1, 1, B, V), lambda i: (0, 0, 0, i, 0)),
            pl.BlockSpec((1, 1, 1, B, V), lambda i: (0, 1, 0, i, 0)),
            pl.BlockSpec((B, 2), lambda i: (i, 0)),
            pl.BlockSpec((B, tr.shape[1]), lambda i: (i, 0)),
            pl.BlockSpec(w0.shape, lambda i: (0, 0)),
            pl.BlockSpec(w1.shape, lambda i: (0, 0)),
            pl.BlockSpec(w2.shape, lambda i: (0, 0)),
            pl.BlockSpec(w_msg.shape, lambda i: (0, 0)),
        ],
        out_specs=[
            pl.BlockSpec((B, CH), lambda i: (i, 0)),
            pl.BlockSpec((B, CH), lambda i: (i, 0)),
        ],
        out_shape=[
            jax.ShapeDtypeStruct((N_DATA, CH), _F32),
            jax.ShapeDtypeStruct((N_DATA, CH), _F32),
        ],
    )(x, x, coords, tr, w0, w1, w2, w_msg)


def _k1b_body(h_ref, wd_ref, hd_ref):
    hd_ref[...] = _dot(jnp.maximum(h_ref[...], 0.0), wd_ref[...])


def _k1b(h, w_dec_dst):
    B = 2000
    n = N_DATA // B
    return pl.pallas_call(
        _k1b_body,
        grid=(n,),
        in_specs=[
            pl.BlockSpec((B, CH), lambda i: (i, 0)),
            pl.BlockSpec(w_dec_dst.shape, lambda i: (0, 0)),
        ],
        out_specs=pl.BlockSpec((B, CH), lambda i: (i, 0)),
        out_shape=jax.ShapeDtypeStruct((N_DATA, CH), _F32),
    )(h, w_dec_dst)


def _k2_body(fh_ref, wdst_ref, parts_ref, wu_ref, wm0_ref, xl_ref, m0_ref):
    agg = parts_ref[0] + parts_ref[1]
    hdst = _dot(fh_ref[...], wdst_ref[...])
    xl = jnp.maximum(hdst + _dot(agg, wu_ref[...]), 0.0)
    xl_ref[...] = xl
    m0_ref[...] = jnp.maximum(_dot(xl, wm0_ref[...]), 0.0)


def _k2(fh, w_dst, parts, w_upd, w_msg0):
    B = 1024
    n = N_HID_PAD // B
    return pl.pallas_call(
        _k2_body,
        grid=(n,),
        in_specs=[
            pl.BlockSpec((B, fh.shape[1]), lambda i: (i, 0)),
            pl.BlockSpec(w_dst.shape, lambda i: (0, 0)),
            pl.BlockSpec((2, B, CH), lambda i: (0, i, 0)),
            pl.BlockSpec(w_upd.shape, lambda i: (0, 0)),
            pl.BlockSpec(w_msg0.shape, lambda i: (0, 0)),
        ],
        out_specs=[
            pl.BlockSpec((B, CH), lambda i: (i, 0)),
            pl.BlockSpec((B, CH), lambda i: (i, 0)),
        ],
        out_shape=[
            jax.ShapeDtypeStruct((N_HID_PAD, CH), _F32),
            jax.ShapeDtypeStruct((N_HID_PAD, CH), _F32),
        ],
    )(fh, w_dst, parts, w_upd, w_msg0)


def _k3_body(h_ref, parts_ref, wu_ref, wm_ref, h1_ref, m1_ref):
    agg = parts_ref[0] + parts_ref[1]
    h1 = h_ref[...] + jnp.maximum(_dot(agg, wu_ref[...]), 0.0)
    h1_ref[...] = h1
    m1_ref[...] = jnp.maximum(_dot(h1, wm_ref[...]), 0.0)


def _k3(h, parts, w_upd, w_msg):
    B = 1024
    n = N_HID_PAD // B
    return pl.pallas_call(
        _k3_body,
        grid=(n,),
        in_specs=[
            pl.BlockSpec((B, CH), lambda i: (i, 0)),
            pl.BlockSpec((2, B, CH), lambda i: (0, i, 0)),
            pl.BlockSpec(w_upd.shape, lambda i: (0, 0)),
            pl.BlockSpec(w_msg.shape, lambda i: (0, 0)),
        ],
        out_specs=[
            pl.BlockSpec((B, CH), lambda i: (i, 0)),
            pl.BlockSpec((B, CH), lambda i: (i, 0)),
        ],
        out_shape=[
            jax.ShapeDtypeStruct((N_HID_PAD, CH), _F32),
            jax.ShapeDtypeStruct((N_HID_PAD, CH), _F32),
        ],
    )(h, parts, w_upd, w_msg)


def _k4_body(h1_ref, xl_ref, parts_ref, wu_ref, wmd_ref, md_ref):
    agg = parts_ref[0] + parts_ref[1]
    xlp = h1_ref[...] + jnp.maximum(_dot(agg, wu_ref[...]), 0.0) + xl_ref[...]
    md_ref[...] = jnp.maximum(_dot(xlp, wmd_ref[...]), 0.0)


def _k4(h1, xl, parts, w_upd, w_msg_dec):
    B = 1024
    n = N_HID_PAD // B
    return pl.pallas_call(
        _k4_body,
        grid=(n,),
        in_specs=[
            pl.BlockSpec((B, CH), lambda i: (i, 0)),
            pl.BlockSpec((B, CH), lambda i: (i, 0)),
            pl.BlockSpec((2, B, CH), lambda i: (0, i, 0)),
            pl.BlockSpec(w_upd.shape, lambda i: (0, 0)),
            pl.BlockSpec(w_msg_dec.shape, lambda i: (0, 0)),
        ],
        out_specs=pl.BlockSpec((B, CH), lambda i: (i, 0)),
        out_shape=jax.ShapeDtypeStruct((N_HID_PAD, CH), _F32),
    )(h1, xl, parts, w_upd, w_msg_dec)


def _k5_body(hd_ref, a_ref, wu_ref, wo_ref, x1_ref, out_ref):
    hf = jnp.maximum(hd_ref[...] + _dot(a_ref[...], wu_ref[...]), 0.0)
    out_ref[...] = _dot(hf, wo_ref[...]) + x1_ref[0, 0, 0]


def _k5(hd, a, w_upd, w_out, x):
    B = 2000
    n = N_DATA // B
    V = x.shape[-1]
    return pl.pallas_call(
        _k5_body,
        grid=(n,),
        in_specs=[
            pl.BlockSpec((B, CH), lambda i: (i, 0)),
            pl.BlockSpec((B, CH), lambda i: (i, 0)),
            pl.BlockSpec(w_upd.shape, lambda i: (0, 0)),
            pl.BlockSpec(w_out.shape, lambda i: (0, 0)),
            pl.BlockSpec((1, 1, 1, B, V), lambda i: (0, 1, 0, i, 0)),
        ],
        out_specs=pl.BlockSpec((B, C_OUT), lambda i: (i, 0)),
        out_shape=jax.ShapeDtypeStruct((N_DATA, C_OUT), _F32),
    )(hd, a, w_upd, w_out, x)


_EB = 128
_CPT = 80
_PP = 16
_EPAD = 32 * _CPT * _EB
_NHA = 10368
_TPH = _NHA // 16


def _sc_agg_hidden(p_hbm, src2d, dst2d, zeros_hid):
    mesh = plsc.VectorSubcoreMesh(core_axis_name="c", subcore_axis_name="s")

    @functools.partial(
        pl.kernel,
        out_type=jax.ShapeDtypeStruct((2, _NHA, CH), _F32),
        mesh=mesh,
        compiler_params=pltpu.CompilerParams(needs_layout_passes=False),
        scratch_types=[
            pltpu.VMEM((_PP, _EB), jnp.int32),
            pltpu.VMEM((_PP, _EB), jnp.int32),
            pltpu.VMEM((_EB, CH), _F32),
            pltpu.VMEM((_EB, CH), _F32),
            pltpu.VMEM_SHARED((_NHA, CH), _F32),
            pltpu.SemaphoreType.DMA,
            pltpu.SemaphoreType.DMA,
        ],
    )
    def k(p_ref, src_ref, dst_ref, z_ref, out_ref,
          srcv, dstv, rows0, rows1, acc, sem0, sem1):
        c = lax.axis_index("c")
        s = lax.axis_index("s")
        wid = c * 16 + s

        pltpu.sync_copy(z_ref, acc.at[pl.ds(s * _TPH, _TPH)])
        plsc.subcore_barrier()

        def piece(p, _):
            base = wid * _CPT + p * _PP
            pltpu.sync_copy(src_ref.at[pl.ds(base, _PP)], srcv)
            pltpu.sync_copy(dst_ref.at[pl.ds(base, _PP)], dstv)

            pltpu.async_copy(p_ref.at[srcv.at[0]], rows0, sem0)

            def pair(t2, _):
                jj = 2 * t2
                pltpu.make_async_copy(p_ref.at[srcv.at[0]], rows0,
                                      sem0).wait()
                pltpu.async_copy(p_ref.at[srcv.at[jj + 1]], rows1, sem1)
                pltpu.sync_copy(rows0, acc.at[dstv.at[jj]], add=True)
                pltpu.make_async_copy(p_ref.at[srcv.at[0]], rows1,
                                      sem1).wait()

                @pl.when(jj + 2 < _PP)
                def _():
                    pltpu.async_copy(
                        p_ref.at[srcv.at[jnp.minimum(jj + 2, _PP - 1)]],
                        rows0, sem0)

                pltpu.sync_copy(rows1, acc.at[dstv.at[jj + 1]], add=True)
                return 0

            lax.fori_loop(0, _PP // 2, pair, 0)
            return 0

        lax.fori_loop(0, _CPT // _PP, piece, 0)
        plsc.subcore_barrier()
        pltpu.sync_copy(acc.at[pl.ds(s * _TPH, _TPH)],
                        out_ref.at[c, pl.ds(s * _TPH, _TPH)])

    return k(p_hbm, src2d, dst2d, zeros_hid)


_CDST = 10240
_CPAD = 10368
_NCHUNK = 10
_SH = E // 16
_ES = 2000
_BT = 128
_CAP = 3200
_TPR = _CPAD // 16
_SHIFT = 14
_MASKV = (1 << _SHIFT) - 1


def _sc_agg_data(p_hbm, src_hbm, dst_hbm, zeros_dat):
    mesh = plsc.VectorSubcoreMesh(core_axis_name="c", subcore_axis_name="s")

    @functools.partial(
        pl.kernel,
        out_type=jax.ShapeDtypeStruct((_NCHUNK * _CDST, CH), _F32),
        mesh=mesh,
        compiler_params=pltpu.CompilerParams(needs_layout_passes=False),
        scratch_types=[
            pltpu.VMEM((_ES,), jnp.int32),
            pltpu.VMEM((_ES,), jnp.int32),
            pltpu.VMEM((_CAP,), jnp.int32),
            pltpu.VMEM((_BT,), jnp.int32),
            pltpu.VMEM((_BT,), jnp.int32),
            pltpu.VMEM((_BT,), jnp.int32),
            pltpu.VMEM((_BT,), jnp.int32),
            pltpu.VMEM((_BT, CH), _F32),
            pltpu.VMEM((_BT, CH), _F32),
            pltpu.VMEM_SHARED((_CPAD, CH), _F32),
            pltpu.SemaphoreType.DMA,
            pltpu.SemaphoreType.DMA,
        ],
    )
    def k(p_ref, src_ref, dst_ref, z_ref, out_ref,
          stag_s, stag_d, cpk, bs0, bd0, bs1, bd1, rows0, rows1, acc,
          sem0, sem1):
        c = lax.axis_index("c")
        s = lax.axis_index("s")

        def prep(j, bs, bd, n):
            for i in range(_BT // 16):
                v = cpk[pl.ds(j * _BT + i * 16, 16)]
                sv = v & _MASKV
                dv = lax.shift_right_logical(v, _SHIFT)
                if n is not None:
                    valid = (j * _BT + i * 16 + lax.iota(jnp.int32, 16)) < n
                    sv = jnp.where(valid, sv, 0)
                    dv = jnp.where(valid, dv, _CDST)
                bs[pl.ds(i * 16, 16)] = sv
                bd[pl.ds(i * 16, 16)] = dv

        def do_batches(nb, n=None):
            @pl.when(nb > 0)
            def _():
                prep(0, bs0, bd0, n)
                pltpu.async_copy(p_ref.at[bs0], rows0, sem0)

            def pair(tt, _):
                j = 2 * tt
                pltpu.make_async_copy(p_ref.at[bs0], rows0, sem0).wait()

                @pl.when(j + 1 < nb)
                def _():
                    prep(j + 1, bs1, bd1, n)
                    pltpu.async_copy(p_ref.at[bs1], rows1, sem1)

                pltpu.sync_copy(rows0, acc.at[bd0], add=True)

                @pl.when(j + 1 < nb)
                def _():
                    pltpu.make_async_copy(p_ref.at[bs1], rows1, sem1).wait()

                    @pl.when(j + 2 < nb)
                    def _():
                        prep(j + 2, bs0, bd0, n)
                        pltpu.async_copy(p_ref.at[bs0], rows0, sem0)

                    pltpu.sync_copy(rows1, acc.at[bd1], add=True)

                return 0

            lax.fori_loop(0, (nb + 1) // 2, pair, 0)

        for ci in range(_NCHUNK // 2):
            chunk = c * (_NCHUNK // 2) + ci
            lo = chunk * _CDST

            pltpu.sync_copy(z_ref, acc.at[pl.ds(s * _TPR, _TPR)])
            plsc.subcore_barrier()

            zc = jnp.zeros((16,), jnp.int32)

            def piece(p, cnt):
                off = s * _SH + p * _ES
                pltpu.sync_copy(src_ref.at[pl.ds(off, _ES)], stag_s)
                pltpu.sync_copy(dst_ref.at[pl.ds(off, _ES)], stag_d)

                def inner(i, cnt):
                    d = stag_d[pl.ds(i * 16, 16)]
                    sv = stag_s[pl.ds(i * 16, 16)]
                    m = (d >= lo) & (d < lo + _CDST)
                    mi = m.astype(jnp.int32)
                    pos = cnt + plsc.cumsum(mi) - mi
                    idx = jnp.where(m, pos, _CAP - 1)
                    v = lax.shift_left(d - lo, _SHIFT) | sv
                    plsc.store_scatter(cpk, [idx], v)
                    return cnt + plsc.all_reduce_population_count(m)

                cnt = lax.fori_loop(0, _ES // 16, inner, cnt)
                n = jnp.max(cnt)
                nb = n // _BT
                nb = jnp.where(nb >= 8, nb, 0)
                do_batches(nb)
                for i in range(_BT // 16):
                    tv_ = cpk[pl.ds(nb * _BT + i * 16, 16)]
                    cpk[pl.ds(i * 16, 16)] = tv_
                return cnt - nb * _BT

            cnt = lax.fori_loop(0, _SH // _ES, piece, zc)
            n = jnp.max(cnt)

            do_batches((n + _BT - 1) // _BT, n)
            plsc.subcore_barrier()

            r0 = s * (_CDST // 16)
            pltpu.sync_copy(acc.at[pl.ds(r0, _CDST // 16)],
                            out_ref.at[pl.ds(chunk * _CDST + r0, _CDST // 16)])

    return k(p_hbm, src_hbm, dst_hbm, zeros_dat)


def kernel(x, coords_data, coords_hidden, trainable_data, trainable_hidden,
           W_enc_src, W_enc_dst, W_enc_msg, W_enc_upd,
           W_proc_msg_0, W_proc_upd_0, W_proc_msg_1, W_proc_upd_1,
           W_dec_dst, W_dec_msg, W_dec_upd, W_out,
           enc_src_idx, enc_dst_idx, proc_src_idx, proc_dst_idx,
           dec_src_idx, dec_dst_idx):
    b, t, e, g, v = x.shape
    latlons_hidden = jnp.concatenate(
        [jnp.sin(coords_hidden), jnp.cos(coords_hidden)], axis=-1)
    fh = jnp.concatenate([latlons_hidden, trainable_hidden], axis=-1)
    fh = jnp.pad(fh, ((0, N_HID_PAD - N_HID), (0, 0)))

    h_src, msg_enc = _k1a(x, coords_data, trainable_data,
                          W_enc_src[:v], W_enc_src[v:2 * v],
                          W_enc_src[2 * v:], W_enc_msg)
    hd = _k1b(h_src, W_dec_dst)

    zeros_hid = jnp.zeros((_TPH, CH), _F32)
    zeros_dat = jnp.zeros((_TPR, CH), _F32)

    npad = _EPAD - E
    pad_src = jnp.arange(npad, dtype=jnp.int32) % N_HID
    pad_dst = N_HID_PAD + jnp.arange(npad, dtype=jnp.int32) % (_NHA - N_HID_PAD)

    def _pad_idx(idx, padv):
        return jnp.concatenate([idx, padv]).reshape(-1, _EB)

    parts_enc = _sc_agg_hidden(
        msg_enc, _pad_idx(enc_src_idx, pad_src),
        _pad_idx(enc_dst_idx, pad_dst), zeros_hid)

    xl, msg_p0 = _k2(fh, W_enc_dst, parts_enc, W_enc_upd, W_proc_msg_0)

    src2d_p = _pad_idx(proc_src_idx, pad_src)
    dst2d_p = _pad_idx(proc_dst_idx, pad_dst)

    parts0 = _sc_agg_hidden(msg_p0, src2d_p, dst2d_p, zeros_hid)
    h1, msg_p1 = _k3(xl, parts0, W_proc_upd_0, W_proc_msg_1)

    parts1 = _sc_agg_hidden(msg_p1, src2d_p, dst2d_p, zeros_hid)
    msg_dec = _k4(h1, xl, parts1, W_proc_upd_1, W_dec_msg)

    a = _sc_agg_data(msg_dec, dec_src_idx, dec_dst_idx, zeros_dat)

    out = _k5(hd, a, W_dec_upd, W_out, x)
    return out.reshape(b, e, g, C_OUT)


# speedup vs baseline: 1.2062x; 1.1921x over previous
import functools

import jax
import jax.numpy as jnp
from jax import lax
from jax.experimental import pallas as pl
from jax.experimental.pallas import tpu as pltpu
from jax.experimental.pallas import tpu_sc as plsc

N_DATA = 100000
N_HID = 10000
N_HID_PAD = 10240
E = 320000
CH = 128
C_OUT = 80


_F32 = jnp.float32


def _dot(a, b):
    return jnp.dot(a, b, preferred_element_type=_F32)


def _k1a_body(x0_ref, x1_ref, c_ref, tr_ref, w0_ref, w1_ref, w2_ref,
              wm_ref, h_ref, msg_ref):
    x0 = x0_ref[0, 0, 0]
    x1 = x1_ref[0, 0, 0]
    c = c_ref[...]
    f12 = jnp.concatenate([jnp.sin(c), jnp.cos(c), tr_ref[...]], axis=-1)
    h = (_dot(x0, w0_ref[...]) + _dot(x1, w1_ref[...])
         + _dot(f12, w2_ref[...]))
    h_ref[...] = h
    msg_ref[...] = jnp.maximum(_dot(h, wm_ref[...]), 0.0)


def _k1a(x, coords, tr, w0, w1, w2, w_msg):
    B = 2000
    n = N_DATA // B
    V = x.shape[-1]
    return pl.pallas_call(
        _k1a_body,
        grid=(n,),
        in_specs=[
            pl.BlockSpec((1, 1, 1, B, V), lambda i: (0, 0, 0, i, 0)),
            pl.BlockSpec((1, 1, 1, B, V), lambda i: (0, 1, 0, i, 0)),
            pl.BlockSpec((B, 2), lambda i: (i, 0)),
            pl.BlockSpec((B, tr.shape[1]), lambda i: (i, 0)),
            pl.BlockSpec(w0.shape, lambda i: (0, 0)),
            pl.BlockSpec(w1.shape, lambda i: (0, 0)),
            pl.BlockSpec(w2.shape, lambda i: (0, 0)),
            pl.BlockSpec(w_msg.shape, lambda i: (0, 0)),
        ],
        out_specs=[
            pl.BlockSpec((B, CH), lambda i: (i, 0)),
            pl.BlockSpec((B, CH), lambda i: (i, 0)),
        ],
        out_shape=[
            jax.ShapeDtypeStruct((N_DATA, CH), _F32),
            jax.ShapeDtypeStruct((N_DATA, CH), _F32),
        ],
    )(x, x, coords, tr, w0, w1, w2, w_msg)


def _k1b_body(h_ref, wd_ref, hd_ref):
    hd_ref[...] = _dot(jnp.maximum(h_ref[...], 0.0), wd_ref[...])


def _k1b(h, w_dec_dst):
    B = 2000
    n = N_DATA // B
    return pl.pallas_call(
        _k1b_body,
        grid=(n,),
        in_specs=[
            pl.BlockSpec((B, CH), lambda i: (i, 0)),
            pl.BlockSpec(w_dec_dst.shape, lambda i: (0, 0)),
        ],
        out_specs=pl.BlockSpec((B, CH), lambda i: (i, 0)),
        out_shape=jax.ShapeDtypeStruct((N_DATA, CH), _F32),
    )(h, w_dec_dst)


def _k2_body(fh_ref, wdst_ref, parts_ref, wu_ref, wm0_ref, xl_ref, m0_ref):
    agg = parts_ref[0] + parts_ref[1]
    hdst = _dot(fh_ref[...], wdst_ref[...])
    xl = jnp.maximum(hdst + _dot(agg, wu_ref[...]), 0.0)
    xl_ref[...] = xl
    m0_ref[...] = jnp.maximum(_dot(xl, wm0_ref[...]), 0.0)


def _k2(fh, w_dst, parts, w_upd, w_msg0):
    B = 1024
    n = N_HID_PAD // B
    return pl.pallas_call(
        _k2_body,
        grid=(n,),
        in_specs=[
            pl.BlockSpec((B, fh.shape[1]), lambda i: (i, 0)),
            pl.BlockSpec(w_dst.shape, lambda i: (0, 0)),
            pl.BlockSpec((2, B, CH), lambda i: (0, i, 0)),
            pl.BlockSpec(w_upd.shape, lambda i: (0, 0)),
            pl.BlockSpec(w_msg0.shape, lambda i: (0, 0)),
        ],
        out_specs=[
            pl.BlockSpec((B, CH), lambda i: (i, 0)),
            pl.BlockSpec((B, CH), lambda i: (i, 0)),
        ],
        out_shape=[
            jax.ShapeDtypeStruct((N_HID_PAD, CH), _F32),
            jax.ShapeDtypeStruct((N_HID_PAD, CH), _F32),
        ],
    )(fh, w_dst, parts, w_upd, w_msg0)


def _k3_body(h_ref, parts_ref, wu_ref, wm_ref, h1_ref, m1_ref):
    agg = parts_ref[0] + parts_ref[1]
    h1 = h_ref[...] + jnp.maximum(_dot(agg, wu_ref[...]), 0.0)
    h1_ref[...] = h1
    m1_ref[...] = jnp.maximum(_dot(h1, wm_ref[...]), 0.0)


def _k3(h, parts, w_upd, w_msg):
    B = 1024
    n = N_HID_PAD // B
    return pl.pallas_call(
        _k3_body,
        grid=(n,),
        in_specs=[
            pl.BlockSpec((B, CH), lambda i: (i, 0)),
            pl.BlockSpec((2, B, CH), lambda i: (0, i, 0)),
            pl.BlockSpec(w_upd.shape, lambda i: (0, 0)),
            pl.BlockSpec(w_msg.shape, lambda i: (0, 0)),
        ],
        out_specs=[
            pl.BlockSpec((B, CH), lambda i: (i, 0)),
            pl.BlockSpec((B, CH), lambda i: (i, 0)),
        ],
        out_shape=[
            jax.ShapeDtypeStruct((N_HID_PAD, CH), _F32),
            jax.ShapeDtypeStruct((N_HID_PAD, CH), _F32),
        ],
    )(h, parts, w_upd, w_msg)


def _k4_body(h1_ref, xl_ref, parts_ref, wu_ref, wmd_ref, md_ref):
    agg = parts_ref[0] + parts_ref[1]
    xlp = h1_ref[...] + jnp.maximum(_dot(agg, wu_ref[...]), 0.0) + xl_ref[...]
    md_ref[...] = jnp.maximum(_dot(xlp, wmd_ref[...]), 0.0)


def _k4(h1, xl, parts, w_upd, w_msg_dec):
    B = 1024
    n = N_HID_PAD // B
    return pl.pallas_call(
        _k4_body,
        grid=(n,),
        in_specs=[
            pl.BlockSpec((B, CH), lambda i: (i, 0)),
            pl.BlockSpec((B, CH), lambda i: (i, 0)),
            pl.BlockSpec((2, B, CH), lambda i: (0, i, 0)),
            pl.BlockSpec(w_upd.shape, lambda i: (0, 0)),
            pl.BlockSpec(w_msg_dec.shape, lambda i: (0, 0)),
        ],
        out_specs=pl.BlockSpec((B, CH), lambda i: (i, 0)),
        out_shape=jax.ShapeDtypeStruct((N_HID_PAD, CH), _F32),
    )(h1, xl, parts, w_upd, w_msg_dec)


def _k5_body(hd_ref, a_ref, wu_ref, wo_ref, x1_ref, out_ref):
    hf = jnp.maximum(hd_ref[...] + _dot(a_ref[...], wu_ref[...]), 0.0)
    out_ref[...] = _dot(hf, wo_ref[...]) + x1_ref[0, 0, 0]


def _k5(hd, a, w_upd, w_out, x):
    B = 2000
    n = N_DATA // B
    V = x.shape[-1]
    return pl.pallas_call(
        _k5_body,
        grid=(n,),
        in_specs=[
            pl.BlockSpec((B, CH), lambda i: (i, 0)),
            pl.BlockSpec((B, CH), lambda i: (i, 0)),
            pl.BlockSpec(w_upd.shape, lambda i: (0, 0)),
            pl.BlockSpec(w_out.shape, lambda i: (0, 0)),
            pl.BlockSpec((1, 1, 1, B, V), lambda i: (0, 1, 0, i, 0)),
        ],
        out_specs=pl.BlockSpec((B, C_OUT), lambda i: (i, 0)),
        out_shape=jax.ShapeDtypeStruct((N_DATA, C_OUT), _F32),
    )(hd, a, w_upd, w_out, x)


_EB = 128
_CPT = 80
_PP = 16
_EPAD = 32 * _CPT * _EB
_NHA = 10368
_TPH = _NHA // 16


def _sc_agg_hidden(p_hbm, src2d, dst2d, zeros_hid):
    mesh = plsc.VectorSubcoreMesh(core_axis_name="c", subcore_axis_name="s")

    @functools.partial(
        pl.kernel,
        out_type=jax.ShapeDtypeStruct((2, _NHA, CH), _F32),
        mesh=mesh,
        compiler_params=pltpu.CompilerParams(needs_layout_passes=False),
        scratch_types=[
            pltpu.VMEM((_PP, _EB), jnp.int32),
            pltpu.VMEM((_PP, _EB), jnp.int32),
            pltpu.VMEM((_EB, CH), _F32),
            pltpu.VMEM((_EB, CH), _F32),
            pltpu.VMEM_SHARED((_NHA, CH), _F32),
            pltpu.SemaphoreType.DMA,
            pltpu.SemaphoreType.DMA,
        ],
    )
    def k(p_ref, src_ref, dst_ref, z_ref, out_ref,
          srcv, dstv, rows0, rows1, acc, sem0, sem1):
        c = lax.axis_index("c")
        s = lax.axis_index("s")
        wid = c * 16 + s

        pltpu.sync_copy(z_ref, acc.at[pl.ds(s * _TPH, _TPH)])
        plsc.subcore_barrier()

        def piece(p, _):
            base = wid * _CPT + p * _PP
            pltpu.sync_copy(src_ref.at[pl.ds(base, _PP)], srcv)
            pltpu.sync_copy(dst_ref.at[pl.ds(base, _PP)], dstv)

            pltpu.async_copy(p_ref.at[srcv.at[0]], rows0, sem0)

            def pair(t2, _):
                jj = 2 * t2
                pltpu.make_async_copy(p_ref.at[srcv.at[0]], rows0,
                                      sem0).wait()
                pltpu.async_copy(p_ref.at[srcv.at[jj + 1]], rows1, sem1)
                pltpu.sync_copy(rows0, acc.at[dstv.at[jj]], add=True)
                pltpu.make_async_copy(p_ref.at[srcv.at[0]], rows1,
                                      sem1).wait()

                @pl.when(jj + 2 < _PP)
                def _():
                    pltpu.async_copy(
                        p_ref.at[srcv.at[jnp.minimum(jj + 2, _PP - 1)]],
                        rows0, sem0)

                pltpu.sync_copy(rows1, acc.at[dstv.at[jj + 1]], add=True)
                return 0

            lax.fori_loop(0, _PP // 2, pair, 0)
            return 0

        lax.fori_loop(0, _CPT // _PP, piece, 0)
        plsc.subcore_barrier()
        pltpu.sync_copy(acc.at[pl.ds(s * _TPH, _TPH)],
                        out_ref.at[c, pl.ds(s * _TPH, _TPH)])

    return k(p_hbm, src2d, dst2d, zeros_hid)


_CDST = 12800
_CPAD = 12928
_NCHUNK = 8
_SH = E // 16
_ES = 2000
_BT = 64
_CAP = 2624
_TPR = _CPAD // 16
_SHIFT = 14
_MASKV = (1 << _SHIFT) - 1


def _sc_agg_data(p_hbm, src_hbm, dst_hbm, zeros_dat):
    mesh = plsc.VectorSubcoreMesh(core_axis_name="c", subcore_axis_name="s")

    @functools.partial(
        pl.kernel,
        out_type=jax.ShapeDtypeStruct((_NCHUNK * _CDST, CH), _F32),
        mesh=mesh,
        compiler_params=pltpu.CompilerParams(needs_layout_passes=False),
        scratch_types=[
            pltpu.VMEM((_ES,), jnp.int32),
            pltpu.VMEM((_ES,), jnp.int32),
            pltpu.VMEM((_CAP,), jnp.int32),
            pltpu.VMEM((_BT,), jnp.int32),
            pltpu.VMEM((_BT,), jnp.int32),
            pltpu.VMEM((_BT,), jnp.int32),
            pltpu.VMEM((_BT,), jnp.int32),
            pltpu.VMEM((_BT, CH), _F32),
            pltpu.VMEM((_BT, CH), _F32),
            pltpu.VMEM_SHARED((_CPAD, CH), _F32),
            pltpu.SemaphoreType.DMA,
            pltpu.SemaphoreType.DMA,
        ],
    )
    def k(p_ref, src_ref, dst_ref, z_ref, out_ref,
          stag_s, stag_d, cpk, bs0, bd0, bs1, bd1, rows0, rows1, acc,
          sem0, sem1):
        c = lax.axis_index("c")
        s = lax.axis_index("s")

        def prep(j, bs, bd, n):
            for i in range(_BT // 16):
                v = cpk[pl.ds(j * _BT + i * 16, 16)]
                sv = v & _MASKV
                dv = lax.shift_right_logical(v, _SHIFT)
                if n is not None:
                    valid = (j * _BT + i * 16 + lax.iota(jnp.int32, 16)) < n
                    sv = jnp.where(valid, sv, 0)
                    dv = jnp.where(valid, dv, _CDST)
                bs[pl.ds(i * 16, 16)] = sv
                bd[pl.ds(i * 16, 16)] = dv

        def do_batches(nb, n=None):
            @pl.when(nb > 0)
            def _():
                prep(0, bs0, bd0, n)
                pltpu.async_copy(p_ref.at[bs0], rows0, sem0)

            def pair(tt, _):
                j = 2 * tt
                pltpu.make_async_copy(p_ref.at[bs0], rows0, sem0).wait()

                @pl.when(j + 1 < nb)
                def _():
                    prep(j + 1, bs1, bd1, n)
                    pltpu.async_copy(p_ref.at[bs1], rows1, sem1)

                pltpu.sync_copy(rows0, acc.at[bd0], add=True)

                @pl.when(j + 1 < nb)
                def _():
                    pltpu.make_async_copy(p_ref.at[bs1], rows1, sem1).wait()

                    @pl.when(j + 2 < nb)
                    def _():
                        prep(j + 2, bs0, bd0, n)
                        pltpu.async_copy(p_ref.at[bs0], rows0, sem0)

                    pltpu.sync_copy(rows1, acc.at[bd1], add=True)

                return 0

            lax.fori_loop(0, (nb + 1) // 2, pair, 0)

        for ci in range(_NCHUNK // 2):
            chunk = c * (_NCHUNK // 2) + ci
            lo = chunk * _CDST

            pltpu.sync_copy(z_ref, acc.at[pl.ds(s * _TPR, _TPR)])
            plsc.subcore_barrier()

            zc = jnp.zeros((16,), jnp.int32)

            def piece(p, cnt):
                off = s * _SH + p * _ES
                pltpu.sync_copy(src_ref.at[pl.ds(off, _ES)], stag_s)
                pltpu.sync_copy(dst_ref.at[pl.ds(off, _ES)], stag_d)

                def inner(i, cnt):
                    d = stag_d[pl.ds(i * 16, 16)]
                    sv = stag_s[pl.ds(i * 16, 16)]
                    m = (d >= lo) & (d < lo + _CDST)
                    mi = m.astype(jnp.int32)
                    pos = cnt + plsc.cumsum(mi) - mi
                    idx = jnp.where(m, pos, _CAP - 1)
                    v = lax.shift_left(d - lo, _SHIFT) | sv
                    plsc.store_scatter(cpk, [idx], v)
                    return cnt + plsc.all_reduce_population_count(m)

                cnt = lax.fori_loop(0, _ES // 16, inner, cnt)
                n = jnp.max(cnt)
                nb = n // _BT
                nb = jnp.where(nb >= 8, nb, 0)
                do_batches(nb)
                for i in range(_BT // 16):
                    tv_ = cpk[pl.ds(nb * _BT + i * 16, 16)]
                    cpk[pl.ds(i * 16, 16)] = tv_
                return cnt - nb * _BT

            cnt = lax.fori_loop(0, _SH // _ES, piece, zc)
            n = jnp.max(cnt)

            do_batches((n + _BT - 1) // _BT, n)
            plsc.subcore_barrier()

            r0 = s * (_CDST // 16)
            pltpu.sync_copy(acc.at[pl.ds(r0, _CDST // 16)],
                            out_ref.at[pl.ds(chunk * _CDST + r0, _CDST // 16)])

    return k(p_hbm, src_hbm, dst_hbm, zeros_dat)


def kernel(x, coords_data, coords_hidden, trainable_data, trainable_hidden,
           W_enc_src, W_enc_dst, W_enc_msg, W_enc_upd,
           W_proc_msg_0, W_proc_upd_0, W_proc_msg_1, W_proc_upd_1,
           W_dec_dst, W_dec_msg, W_dec_upd, W_out,
           enc_src_idx, enc_dst_idx, proc_src_idx, proc_dst_idx,
           dec_src_idx, dec_dst_idx):
    b, t, e, g, v = x.shape
    latlons_hidden = jnp.concatenate(
        [jnp.sin(coords_hidden), jnp.cos(coords_hidden)], axis=-1)
    fh = jnp.concatenate([latlons_hidden, trainable_hidden], axis=-1)
    fh = jnp.pad(fh, ((0, N_HID_PAD - N_HID), (0, 0)))

    h_src, msg_enc = _k1a(x, coords_data, trainable_data,
                          W_enc_src[:v], W_enc_src[v:2 * v],
                          W_enc_src[2 * v:], W_enc_msg)
    hd = _k1b(h_src, W_dec_dst)

    zeros_hid = jnp.zeros((_TPH, CH), _F32)
    zeros_dat = jnp.zeros((_TPR, CH), _F32)

    npad = _EPAD - E
    pad_src = jnp.arange(npad, dtype=jnp.int32) % N_HID
    pad_dst = N_HID_PAD + jnp.arange(npad, dtype=jnp.int32) % (_NHA - N_HID_PAD)

    def _pad_idx(idx, padv):
        return jnp.concatenate([idx, padv]).reshape(-1, _EB)

    parts_enc = _sc_agg_hidden(
        msg_enc, _pad_idx(enc_src_idx, pad_src),
        _pad_idx(enc_dst_idx, pad_dst), zeros_hid)

    xl, msg_p0 = _k2(fh, W_enc_dst, parts_enc, W_enc_upd, W_proc_msg_0)

    src2d_p = _pad_idx(proc_src_idx, pad_src)
    dst2d_p = _pad_idx(proc_dst_idx, pad_dst)

    parts0 = _sc_agg_hidden(msg_p0, src2d_p, dst2d_p, zeros_hid)
    h1, msg_p1 = _k3(xl, parts0, W_proc_upd_0, W_proc_msg_1)

    parts1 = _sc_agg_hidden(msg_p1, src2d_p, dst2d_p, zeros_hid)
    msg_dec = _k4(h1, xl, parts1, W_proc_upd_1, W_dec_msg)

    a = _sc_agg_data(msg_dec, dec_src_idx, dec_dst_idx, zeros_dat)

    out = _k5(hd, a, W_dec_upd, W_out, x)
    return out.reshape(b, e, g, C_OUT)
